# baseline (speedup 1.0000x reference)
import functools

import jax
import jax.numpy as jnp
from jax import lax
from jax.experimental import pallas as pl
from jax.experimental.pallas import tpu as pltpu
from jax.experimental.pallas import tpu_sc as plsc

D = 128
N_MOD = 26
LANES = 16
N_WORKERS = 32


def _sc_body(nchunks, chunk, x_hbm, idx_hbm, types_hbm, emb_hbm, pe_hbm,
             w_hbm, out_hbm, xbuf, idxbuf, tbuf, embbuf, pebuf, wbuf):
    wid = lax.axis_index("s") * 2 + lax.axis_index("c")
    rows_per_worker = nchunks * chunk
    base0 = wid * rows_per_worker

    pltpu.sync_copy(emb_hbm, embbuf)
    pltpu.sync_copy(pe_hbm, pebuf)
    pltpu.sync_copy(w_hbm, wbuf)

    wv = [[wbuf[k, pl.ds(16 * j, LANES)] for j in range(D // LANES)]
          for k in range(4)]
    lane = lax.iota(jnp.int32, 16)

    kk = chunk // N_MOD

    def chunk_body(c, carry):
        base = base0 + c * chunk
        pltpu.sync_copy(x_hbm.at[pl.ds(base, chunk)], xbuf)
        pltpu.sync_copy(idx_hbm.at[pl.ds(base, chunk)], idxbuf)
        pltpu.sync_copy(types_hbm.at[pl.ds(base * 4, chunk * 4)], tbuf)

        def m_body(m, carry_m):
            pe_v = [pebuf[m, pl.ds(16 * j, LANES)] for j in range(D // LANES)]

            def k_body(k, carry_k):
                r = m + N_MOD * k
                idxv = plsc.load_gather(
                    idxbuf, [jnp.full((LANES,), r, jnp.int32)])
                t = [plsc.load_gather(
                        tbuf, [jnp.full((LANES,), 4 * r + q, jnp.int32)])
                     for q in range(4)]
                for j in range(D // LANES):
                    g = plsc.load_gather(embbuf, [idxv, lane + 16 * j])
                    acc = g + pe_v[j]
                    acc = acc + t[0] * wv[0][j]
                    acc = acc + t[1] * wv[1][j]
                    acc = acc + t[2] * wv[2][j]
                    acc = acc + t[3] * wv[3][j]
                    plsc.addupdate(xbuf.at[r, pl.ds(16 * j, LANES)], acc)
                return carry_k

            lax.fori_loop(0, kk, k_body, 0)
            return carry_m

        lax.fori_loop(0, N_MOD, m_body, 0)
        pltpu.sync_copy(xbuf, out_hbm.at[pl.ds(base, chunk)])
        return carry

    lax.fori_loop(0, nchunks, chunk_body, 0)


def _make_pe_plus(n_mod, d_model, b_type):
    position = jnp.arange(0, n_mod, dtype=jnp.float32)[:, None]
    div_term = jnp.exp(jnp.arange(0, d_model, 2, dtype=jnp.float32)
                       * (-(jnp.log(10000.0) / d_model)))
    pe = jnp.zeros((n_mod, d_model), dtype=jnp.float32)
    pe = pe.at[:, 0::2].set(jnp.sin(position * div_term))
    pe = pe.at[:, 1::2].set(jnp.cos(position * div_term))
    return pe + b_type[None, :]


def kernel(x, modality_indices, modality_types, emb_table, W_type, b_type):
    batch, n_mod, d_model = x.shape
    assert n_mod == N_MOD and d_model == D
    rows = batch * n_mod
    rows_per_worker = rows // N_WORKERS
    assert rows_per_worker * N_WORKERS == rows
    chunk = N_MOD * 8
    nchunks = rows_per_worker // chunk
    assert nchunks * chunk == rows_per_worker

    x2 = x.reshape(rows, d_model)
    idx = modality_indices.reshape(rows).astype(jnp.int32)
    types_flat = modality_types.reshape(rows * 4)
    pe_plus = _make_pe_plus(n_mod, d_model, b_type)

    grid_kernel = pl.kernel(
        functools.partial(_sc_body, nchunks, chunk),
        out_type=jax.ShapeDtypeStruct((rows, d_model), jnp.float32),
        mesh=plsc.VectorSubcoreMesh(core_axis_name="c", subcore_axis_name="s"),
        scratch_types=[
            pltpu.VMEM((chunk, d_model), jnp.float32),
            pltpu.VMEM((chunk,), jnp.int32),
            pltpu.VMEM((chunk * 4,), jnp.float32),
            pltpu.VMEM(emb_table.shape, jnp.float32),
            pltpu.VMEM((n_mod, d_model), jnp.float32),
            pltpu.VMEM((4, d_model), jnp.float32),
        ],
        compiler_params=pltpu.CompilerParams(needs_layout_passes=False),
    )
    out = grid_kernel(x2, idx, types_flat, emb_table, pe_plus, W_type)
    return out.reshape(batch, n_mod, d_model)


# speedup vs baseline: 1.0991x; 1.0991x over previous
import functools

import jax
import jax.numpy as jnp
from jax import lax
from jax.experimental import pallas as pl
from jax.experimental.pallas import tpu as pltpu
from jax.experimental.pallas import tpu_sc as plsc

D = 128
N_MOD = 26
LANES = 16
N_WORKERS = 32
KK = 8
CHUNK = N_MOD * KK


def _sc_body(nchunks, x_hbm, idx_hbm, types_hbm, emb_hbm, pe_hbm,
             w_hbm, out_hbm, xbuf, idxbuf, tbuf, embbuf, pebuf, wbuf,
             sem_in, sem_out):
    wid = lax.axis_index("s") * 2 + lax.axis_index("c")
    base0 = wid * nchunks * CHUNK

    pltpu.sync_copy(emb_hbm, embbuf)
    pltpu.sync_copy(pe_hbm, pebuf)
    pltpu.sync_copy(w_hbm, wbuf)

    wv = [[wbuf[k, pl.ds(16 * j, LANES)] for j in range(D // LANES)]
          for k in range(4)]
    lane = lax.iota(jnp.int32, 16)

    def _copies(c, s):
        base = base0 + c * CHUNK
        return [
            (x_hbm.at[pl.ds(base, CHUNK)],
             xbuf.at[pl.ds(s * CHUNK, CHUNK)]),
            (idx_hbm.at[pl.ds(base, CHUNK)],
             idxbuf.at[pl.ds(s * CHUNK, CHUNK)]),
            (types_hbm.at[pl.ds(base * 4, CHUNK * 4)],
             tbuf.at[pl.ds(s * CHUNK * 4, CHUNK * 4)]),
        ]

    def start_in(c, s):
        for src, dst in _copies(c, s):
            pltpu.async_copy(src, dst, sem_in.at[s])

    def wait_in(c, s):
        for src, dst in _copies(c, s):
            pltpu.make_async_copy(src, dst, sem_in.at[s]).wait()

    def start_out(c, s):
        base = base0 + c * CHUNK
        pltpu.async_copy(xbuf.at[pl.ds(s * CHUNK, CHUNK)],
                         out_hbm.at[pl.ds(base, CHUNK)], sem_out.at[s])

    def wait_out(c, s):
        base = base0 + c * CHUNK
        pltpu.make_async_copy(xbuf.at[pl.ds(s * CHUNK, CHUNK)],
                              out_hbm.at[pl.ds(base, CHUNK)],
                              sem_out.at[s]).wait()

    def compute(s):
        def m_body(m, carry_m):
            pe_v = [pebuf[m, pl.ds(16 * j, LANES)] for j in range(D // LANES)]
            for k in range(KK):
                r = m + N_MOD * k
                idxv = plsc.load_gather(
                    idxbuf, [jnp.full((LANES,), s * CHUNK + r, jnp.int32)])
                t = [plsc.load_gather(
                        tbuf, [jnp.full((LANES,), s * CHUNK * 4 + 4 * r + q,
                                        jnp.int32)])
                     for q in range(4)]
                for j in range(D // LANES):
                    g = plsc.load_gather(embbuf, [idxv, lane + 16 * j])
                    acc = g + pe_v[j]
                    acc = acc + t[0] * wv[0][j]
                    acc = acc + t[1] * wv[1][j]
                    acc = acc + t[2] * wv[2][j]
                    acc = acc + t[3] * wv[3][j]
                    plsc.addupdate(xbuf.at[s * CHUNK + r, pl.ds(16 * j, LANES)], acc)
            return carry_m

        lax.fori_loop(0, N_MOD, m_body, 0)

    start_in(0, 0)

    def chunk_body(c, carry):
        slot = lax.rem(c, 2)
        nxt = 1 - slot

        @pl.when(c + 1 < nchunks)
        def _prefetch():
            @pl.when(c >= 1)
            def _drain_prev_write():
                wait_out(c - 1, nxt)
            start_in(c + 1, nxt)

        wait_in(c, slot)
        compute(slot)
        start_out(c, slot)
        return carry

    lax.fori_loop(0, nchunks, chunk_body, 0)
    wait_out(nchunks - 2, lax.rem(nchunks - 2, 2))
    wait_out(nchunks - 1, lax.rem(nchunks - 1, 2))


def _make_pe_plus(n_mod, d_model, b_type):
    position = jnp.arange(0, n_mod, dtype=jnp.float32)[:, None]
    div_term = jnp.exp(jnp.arange(0, d_model, 2, dtype=jnp.float32)
                       * (-(jnp.log(10000.0) / d_model)))
    pe = jnp.zeros((n_mod, d_model), dtype=jnp.float32)
    pe = pe.at[:, 0::2].set(jnp.sin(position * div_term))
    pe = pe.at[:, 1::2].set(jnp.cos(position * div_term))
    return pe + b_type[None, :]


def kernel(x, modality_indices, modality_types, emb_table, W_type, b_type):
    batch, n_mod, d_model = x.shape
    assert n_mod == N_MOD and d_model == D
    rows = batch * n_mod
    rows_per_worker = rows // N_WORKERS
    assert rows_per_worker * N_WORKERS == rows
    nchunks = rows_per_worker // CHUNK
    assert nchunks * CHUNK == rows_per_worker

    x2 = x.reshape(rows, d_model)
    idx = modality_indices.reshape(rows).astype(jnp.int32)
    types_flat = modality_types.reshape(rows * 4)
    pe_plus = _make_pe_plus(n_mod, d_model, b_type)

    grid_kernel = pl.kernel(
        functools.partial(_sc_body, nchunks),
        out_type=jax.ShapeDtypeStruct((rows, d_model), jnp.float32),
        mesh=plsc.VectorSubcoreMesh(core_axis_name="c", subcore_axis_name="s"),
        scratch_types=[
            pltpu.VMEM((2 * CHUNK, d_model), jnp.float32),
            pltpu.VMEM((2 * CHUNK,), jnp.int32),
            pltpu.VMEM((2 * CHUNK * 4,), jnp.float32),
            pltpu.VMEM(emb_table.shape, jnp.float32),
            pltpu.VMEM((n_mod, d_model), jnp.float32),
            pltpu.VMEM((4, d_model), jnp.float32),
            pltpu.SemaphoreType.DMA((2,)),
            pltpu.SemaphoreType.DMA((2,)),
        ],
        compiler_params=pltpu.CompilerParams(needs_layout_passes=False),
    )
    out = grid_kernel(x2, idx, types_flat, emb_table, pe_plus, W_type)
    return out.reshape(batch, n_mod, d_model)


# speedup vs baseline: 1.9622x; 1.7852x over previous
import functools

import jax
import jax.numpy as jnp
from jax import lax
from jax.experimental import pallas as pl
from jax.experimental.pallas import tpu as pltpu
from jax.experimental.pallas import tpu_sc as plsc

D = 128
N_MOD = 26
LANES = 16
N_WORKERS = 32
KK = 8
CHUNK = N_MOD * KK


def _sc_body(nchunks, x_hbm, idx_hbm, types_hbm, emb_hbm, pe_hbm,
             w_hbm, out_hbm, xbuf, idxbuf, tbuf, embbuf, pebuf, wbuf,
             sem_in, sem_out):
    wid = lax.axis_index("s") * 2 + lax.axis_index("c")
    base0 = wid * nchunks * CHUNK

    pltpu.sync_copy(emb_hbm, embbuf)
    pltpu.sync_copy(pe_hbm, pebuf)
    pltpu.sync_copy(w_hbm, wbuf)

    wv = [[wbuf[k, pl.ds(16 * j, LANES)] for j in range(D // LANES)]
          for k in range(4)]
    lane = lax.iota(jnp.int32, 16)

    def _copies(c, s):
        base = base0 + c * CHUNK
        return [
            (x_hbm.at[pl.ds(base, CHUNK)],
             xbuf.at[pl.ds(s * CHUNK, CHUNK)]),
            (idx_hbm.at[pl.ds(base, CHUNK)],
             idxbuf.at[pl.ds(s * CHUNK, CHUNK)]),
            (types_hbm.at[pl.ds(base * 4, CHUNK * 4)],
             tbuf.at[pl.ds(s * CHUNK * 4, CHUNK * 4)]),
        ]

    def start_in(c, s):
        for src, dst in _copies(c, s):
            pltpu.async_copy(src, dst, sem_in.at[s])

    def wait_in(c, s):
        for src, dst in _copies(c, s):
            pltpu.make_async_copy(src, dst, sem_in.at[s]).wait()

    def start_out(c, s):
        base = base0 + c * CHUNK
        pltpu.async_copy(xbuf.at[pl.ds(s * CHUNK, CHUNK)],
                         out_hbm.at[pl.ds(base, CHUNK)], sem_out.at[s])

    def wait_out(c, s):
        base = base0 + c * CHUNK
        pltpu.make_async_copy(xbuf.at[pl.ds(s * CHUNK, CHUNK)],
                              out_hbm.at[pl.ds(base, CHUNK)],
                              sem_out.at[s]).wait()

    def compute(s):
        def m_body(m, carry_m):
            pe_v = [pebuf[m, pl.ds(16 * j, LANES)] for j in range(D // LANES)]
            for k in range(KK):
                r = m + N_MOD * k
                idxv = plsc.load_gather(
                    idxbuf, [jnp.full((LANES,), s * CHUNK + r, jnp.int32)])
                t = [plsc.load_gather(
                        tbuf, [jnp.full((LANES,), s * CHUNK * 4 + 4 * r + q,
                                        jnp.int32)])
                     for q in range(4)]
                for j in range(D // LANES):
                    g = plsc.load_gather(embbuf, [idxv, lane + 16 * j])
                    acc = g + pe_v[j]
                    acc = acc + t[0] * wv[0][j]
                    acc = acc + t[1] * wv[1][j]
                    acc = acc + t[2] * wv[2][j]
                    acc = acc + t[3] * wv[3][j]
                    plsc.addupdate(xbuf.at[s * CHUNK + r, pl.ds(16 * j, LANES)], acc)
            return carry_m

        lax.fori_loop(0, N_MOD, m_body, 0)

    start_in(0, 0)

    def chunk_body(c, carry):
        slot = lax.rem(c, 2)
        nxt = 1 - slot

        @pl.when(c + 1 < nchunks)
        def _prefetch():
            @pl.when(c >= 1)
            def _drain_prev_write():
                wait_out(c - 1, nxt)
            start_in(c + 1, nxt)

        wait_in(c, slot)
        compute(slot)
        start_out(c, slot)
        return carry

    lax.fori_loop(0, nchunks, chunk_body, 0)
    wait_out(nchunks - 2, lax.rem(nchunks - 2, 2))
    wait_out(nchunks - 1, lax.rem(nchunks - 1, 2))


def _tc_body(x_ref, idx_ref, t_ref, emb_ref, pe_ref, w_ref, o_ref):
    embm = emb_ref[...]
    wm = w_ref[...]
    for m in range(N_MOD):
        idx_m = idx_ref[:, m]
        oh = (idx_m[:, None] == lax.broadcasted_iota(
            jnp.int32, (idx_m.shape[0], 50), 1)).astype(jnp.float32)
        g = lax.dot_general(oh, embm, (((1,), (0,)), ((), ())),
                            preferred_element_type=jnp.float32)
        te = lax.dot_general(t_ref[:, m, :], wm, (((1,), (0,)), ((), ())),
                             preferred_element_type=jnp.float32)
        o_ref[:, m, :] = x_ref[:, m, :] + g + te + pe_ref[m, :][None, :]


def _tc_part(x3, idx2, types3, emb_table, pe_plus, W_type, tb):
    b = x3.shape[0]
    grid = (b // tb,)
    return pl.pallas_call(
        _tc_body,
        grid=grid,
        in_specs=[
            pl.BlockSpec((tb, N_MOD, D), lambda i: (i, 0, 0)),
            pl.BlockSpec((tb, N_MOD), lambda i: (i, 0)),
            pl.BlockSpec((tb, N_MOD, 4), lambda i: (i, 0, 0)),
            pl.BlockSpec((50, D), lambda i: (0, 0)),
            pl.BlockSpec((N_MOD, D), lambda i: (0, 0)),
            pl.BlockSpec((4, D), lambda i: (0, 0)),
        ],
        out_specs=pl.BlockSpec((tb, N_MOD, D), lambda i: (i, 0, 0)),
        out_shape=jax.ShapeDtypeStruct((b, N_MOD, D), jnp.float32),
    )(x3, idx2, types3, emb_table, pe_plus, W_type)


def _make_pe_plus(n_mod, d_model, b_type):
    position = jnp.arange(0, n_mod, dtype=jnp.float32)[:, None]
    div_term = jnp.exp(jnp.arange(0, d_model, 2, dtype=jnp.float32)
                       * (-(jnp.log(10000.0) / d_model)))
    pe = jnp.zeros((n_mod, d_model), dtype=jnp.float32)
    pe = pe.at[:, 0::2].set(jnp.sin(position * div_term))
    pe = pe.at[:, 1::2].set(jnp.cos(position * div_term))
    return pe + b_type[None, :]


B_SC = 4096
TB = 128


def kernel(x, modality_indices, modality_types, emb_table, W_type, b_type):
    batch, n_mod, d_model = x.shape
    assert n_mod == N_MOD and d_model == D
    pe_plus = _make_pe_plus(n_mod, d_model, b_type)
    idx_all = modality_indices.astype(jnp.int32)

    out_sc = _sc_part(x[:B_SC], idx_all[:B_SC], modality_types[:B_SC],
                      emb_table, pe_plus, W_type)
    out_tc = _tc_part(x[B_SC:], idx_all[B_SC:], modality_types[B_SC:],
                      emb_table, pe_plus, W_type, TB)
    return jnp.concatenate([out_sc, out_tc], axis=0)


def _sc_part(x3, idx2, types3, emb_table, pe_plus, W_type):
    batch, n_mod, d_model = x3.shape
    rows = batch * n_mod
    rows_per_worker = rows // N_WORKERS
    assert rows_per_worker * N_WORKERS == rows
    nchunks = rows_per_worker // CHUNK
    assert nchunks * CHUNK == rows_per_worker

    x2 = x3.reshape(rows, d_model)
    idx = idx2.reshape(rows)
    types_flat = types3.reshape(rows * 4)

    grid_kernel = pl.kernel(
        functools.partial(_sc_body, nchunks),
        out_type=jax.ShapeDtypeStruct((rows, d_model), jnp.float32),
        mesh=plsc.VectorSubcoreMesh(core_axis_name="c", subcore_axis_name="s"),
        scratch_types=[
            pltpu.VMEM((2 * CHUNK, d_model), jnp.float32),
            pltpu.VMEM((2 * CHUNK,), jnp.int32),
            pltpu.VMEM((2 * CHUNK * 4,), jnp.float32),
            pltpu.VMEM(emb_table.shape, jnp.float32),
            pltpu.VMEM((n_mod, d_model), jnp.float32),
            pltpu.VMEM((4, d_model), jnp.float32),
            pltpu.SemaphoreType.DMA((2,)),
            pltpu.SemaphoreType.DMA((2,)),
        ],
        compiler_params=pltpu.CompilerParams(needs_layout_passes=False),
    )
    out = grid_kernel(x2, idx, types_flat, emb_table, pe_plus, W_type)
    return out.reshape(batch, n_mod, d_model)


# speedup vs baseline: 2.0554x; 1.0475x over previous
import functools

import jax
import jax.numpy as jnp
from jax import lax
from jax.experimental import pallas as pl
from jax.experimental.pallas import tpu as pltpu
from jax.experimental.pallas import tpu_sc as plsc

D = 128
N_MOD = 26
M_PAD = 32
V = 50
LANES = 16
N_WORKERS = 32
KK = 4
CHUNK = N_MOD * KK
KKM = KK * M_PAD

B_SC = 2048
TB = 128
NSC_BLOCKS = B_SC // TB


def _sc_body(nchunks, x_hbm, idx_hbm, types_hbm, emb_hbm, pe_hbm,
             w_hbm, out_hbm, xbuf, obuf, idxbuf, tbuf, embbuf, pebuf, wbuf,
             sem_in, sem_out):
    wid = lax.axis_index("s") * 2 + lax.axis_index("c")
    base0 = wid * nchunks * CHUNK

    pltpu.sync_copy(emb_hbm, embbuf)
    pltpu.sync_copy(pe_hbm, pebuf)
    pltpu.sync_copy(w_hbm, wbuf)

    wv = [[wbuf[k, pl.ds(16 * j, LANES)] for j in range(D // LANES)]
          for k in range(4)]
    lane = lax.iota(jnp.int32, 16)

    def _in_copies(c, s):
        base = base0 + c * CHUNK
        return [
            (x_hbm.at[pl.ds(base, CHUNK)],
             xbuf.at[pl.ds(s * CHUNK, CHUNK)]),
            (idx_hbm.at[pl.ds(base, CHUNK)],
             idxbuf.at[pl.ds(s * CHUNK, CHUNK)]),
            (types_hbm.at[pl.ds(base * 4, CHUNK * 4)],
             tbuf.at[pl.ds(s * CHUNK * 4, CHUNK * 4)]),
        ]

    def _out_copies(c, s):
        batch0 = (base0 + c * CHUNK) // N_MOD
        return [
            (obuf.at[pl.ds(s * KKM, KKM)],
             out_hbm.at[pl.ds(batch0 * M_PAD, KKM)])
        ]

    def start_in(c, s):
        for src, dst in _in_copies(c, s):
            pltpu.async_copy(src, dst, sem_in.at[s])

    def wait_in(c, s):
        for src, dst in _in_copies(c, s):
            pltpu.make_async_copy(src, dst, sem_in.at[s]).wait()

    def start_out(c, s):
        for src, dst in _out_copies(c, s):
            pltpu.async_copy(src, dst, sem_out.at[s])

    def wait_out(c, s):
        for src, dst in _out_copies(c, s):
            pltpu.make_async_copy(src, dst, sem_out.at[s]).wait()

    def compute(s):
        def m_body(m, carry_m):
            pe_v = [pebuf[m, pl.ds(16 * j, LANES)] for j in range(D // LANES)]
            for k in range(KK):
                r = m + N_MOD * k
                idxv = plsc.load_gather(
                    idxbuf, [jnp.full((LANES,), s * CHUNK + r, jnp.int32)])
                t = [plsc.load_gather(
                        tbuf, [jnp.full((LANES,), s * CHUNK * 4 + 4 * r + q,
                                        jnp.int32)])
                     for q in range(4)]
                for j in range(D // LANES):
                    g = plsc.load_gather(embbuf, [idxv, lane + 16 * j])
                    acc = g + pe_v[j]
                    acc = acc + t[0] * wv[0][j]
                    acc = acc + t[1] * wv[1][j]
                    acc = acc + t[2] * wv[2][j]
                    acc = acc + t[3] * wv[3][j]
                    acc = acc + xbuf[s * CHUNK + r, pl.ds(16 * j, LANES)]
                    obuf[s * KKM + k * M_PAD + m, pl.ds(16 * j, LANES)] = acc
            return carry_m

        lax.fori_loop(0, N_MOD, m_body, 0)

    start_in(0, 0)

    def chunk_body(c, carry):
        slot = lax.rem(c, 2)
        nxt = 1 - slot

        @pl.when(c + 1 < nchunks)
        def _prefetch():
            @pl.when(c >= 1)
            def _drain_prev_write():
                wait_out(c - 1, nxt)
            start_in(c + 1, nxt)

        wait_in(c, slot)
        compute(slot)
        start_out(c, slot)
        return carry

    lax.fori_loop(0, nchunks, chunk_body, 0)
    wait_out(nchunks - 2, lax.rem(nchunks - 2, 2))
    wait_out(nchunks - 1, lax.rem(nchunks - 1, 2))


def _sc_part(x3, idx2, types3, emb_table, pe_plus, W_type):
    batch, n_mod, d_model = x3.shape
    rows = batch * n_mod
    rows_per_worker = rows // N_WORKERS
    assert rows_per_worker * N_WORKERS == rows
    nchunks = rows_per_worker // CHUNK
    assert nchunks * CHUNK == rows_per_worker

    x2 = x3.reshape(rows, d_model)
    idx = idx2.reshape(rows)
    types_flat = types3.reshape(rows * 4)

    grid_kernel = pl.kernel(
        functools.partial(_sc_body, nchunks),
        out_type=jax.ShapeDtypeStruct((batch * M_PAD, d_model), jnp.float32),
        mesh=plsc.VectorSubcoreMesh(core_axis_name="c", subcore_axis_name="s"),
        scratch_types=[
            pltpu.VMEM((2 * CHUNK, d_model), jnp.float32),
            pltpu.VMEM((2 * KKM, d_model), jnp.float32),
            pltpu.VMEM((2 * CHUNK,), jnp.int32),
            pltpu.VMEM((2 * CHUNK * 4,), jnp.float32),
            pltpu.VMEM(emb_table.shape, jnp.float32),
            pltpu.VMEM((n_mod, d_model), jnp.float32),
            pltpu.VMEM((4, d_model), jnp.float32),
            pltpu.SemaphoreType.DMA((2,)),
            pltpu.SemaphoreType.DMA((2,)),
        ],
        compiler_params=pltpu.CompilerParams(needs_layout_passes=False),
    )
    return grid_kernel(x2, idx, types_flat, emb_table, pe_plus, W_type)


def _tc_body(x_ref, idx_ref, tt_ref, emb_ref, pe_ref, w_ref, sc_ref, o_ref):
    i = pl.program_id(0)

    @pl.when(i < NSC_BLOCKS)
    def _passthrough():
        sc = sc_ref[...].reshape(TB, M_PAD, D)
        o_ref[:, 0:24, :] = sc[:, 0:24, :]
        o_ref[:, 24, :] = sc[:, 24, :]
        o_ref[:, 25, :] = sc[:, 25, :]

    @pl.when(i >= NSC_BLOCKS)
    def _compute():
        embm = emb_ref[...]
        wm = w_ref[...]
        for m in range(N_MOD):
            idx_m = idx_ref[:, m]
            oh = (idx_m[:, None] == lax.broadcasted_iota(
                jnp.int32, (TB, V), 1)).astype(jnp.float32)
            g = lax.dot_general(oh, embm, (((1,), (0,)), ((), ())),
                                preferred_element_type=jnp.float32)
            te = lax.dot_general(tt_ref[m], wm, (((1,), (0,)), ((), ())),
                                 preferred_element_type=jnp.float32)
            o_ref[:, m, :] = x_ref[:, m, :] + g + te + pe_ref[m, :][None, :]


def _tc_assemble(x, idx2, types_t, emb_table, pe_plus, W_type, sc_out):
    batch = x.shape[0]
    grid = (batch // TB,)
    nsc = NSC_BLOCKS
    return pl.pallas_call(
        _tc_body,
        grid=grid,
        in_specs=[
            pl.BlockSpec((TB, N_MOD, D), lambda i: (jnp.maximum(i, nsc), 0, 0)),
            pl.BlockSpec((TB, N_MOD), lambda i: (jnp.maximum(i, nsc), 0)),
            pl.BlockSpec((N_MOD, TB, 4), lambda i: (0, jnp.maximum(i, nsc), 0)),
            pl.BlockSpec((V, D), lambda i: (0, 0)),
            pl.BlockSpec((N_MOD, D), lambda i: (0, 0)),
            pl.BlockSpec((4, D), lambda i: (0, 0)),
            pl.BlockSpec((TB * M_PAD, D),
                         lambda i: (jnp.minimum(i, nsc - 1), 0)),
        ],
        out_specs=pl.BlockSpec((TB, N_MOD, D), lambda i: (i, 0, 0)),
        out_shape=jax.ShapeDtypeStruct((batch, N_MOD, D), jnp.float32),
    )(x, idx2, types_t, emb_table, pe_plus, W_type, sc_out)


def _make_pe_plus(n_mod, d_model, b_type):
    position = jnp.arange(0, n_mod, dtype=jnp.float32)[:, None]
    div_term = jnp.exp(jnp.arange(0, d_model, 2, dtype=jnp.float32)
                       * (-(jnp.log(10000.0) / d_model)))
    pe = jnp.zeros((n_mod, d_model), dtype=jnp.float32)
    pe = pe.at[:, 0::2].set(jnp.sin(position * div_term))
    pe = pe.at[:, 1::2].set(jnp.cos(position * div_term))
    return pe + b_type[None, :]


def kernel(x, modality_indices, modality_types, emb_table, W_type, b_type):
    batch, n_mod, d_model = x.shape
    assert n_mod == N_MOD and d_model == D
    pe_plus = _make_pe_plus(n_mod, d_model, b_type)
    idx_all = modality_indices.astype(jnp.int32)
    types_t = jnp.transpose(modality_types, (1, 0, 2))

    sc_out = _sc_part(x[:B_SC], idx_all[:B_SC], modality_types[:B_SC],
                      emb_table, pe_plus, W_type)
    return _tc_assemble(x, idx_all, types_t, emb_table, pe_plus, W_type,
                        sc_out)


# speedup vs baseline: 3.1531x; 1.5341x over previous
import functools

import jax
import jax.numpy as jnp
from jax import lax
from jax.experimental import pallas as pl
from jax.experimental.pallas import tpu as pltpu
from jax.experimental.pallas import tpu_sc as plsc

D = 128
N_MOD = 26
M_PAD = 32
V = 50
LANES = 16
N_WORKERS = 32
KK = 4
CHUNK = N_MOD * KK
KKM = KK * M_PAD

B_SC = 2048
TB = 128
NSC_BLOCKS = B_SC // TB


def _sc_body(nchunks, x_hbm, idx_hbm, types_hbm, emb_hbm, pe_hbm,
             w_hbm, out_hbm, xbuf, obuf, idxbuf, tbuf, embbuf, pebuf, wbuf,
             sem_in, sem_out):
    wid = lax.axis_index("s") * 2 + lax.axis_index("c")
    base0 = wid * nchunks * CHUNK

    pltpu.sync_copy(emb_hbm, embbuf)
    pltpu.sync_copy(pe_hbm, pebuf)
    pltpu.sync_copy(w_hbm, wbuf)

    wv = [[wbuf[k, pl.ds(16 * j, LANES)] for j in range(D // LANES)]
          for k in range(4)]
    lane = lax.iota(jnp.int32, 16)

    def _in_copies(c, s):
        base = base0 + c * CHUNK
        return [
            (x_hbm.at[pl.ds(base, CHUNK)],
             xbuf.at[pl.ds(s * CHUNK, CHUNK)]),
            (idx_hbm.at[pl.ds(base, CHUNK)],
             idxbuf.at[pl.ds(s * CHUNK, CHUNK)]),
            (types_hbm.at[pl.ds(base * 4, CHUNK * 4)],
             tbuf.at[pl.ds(s * CHUNK * 4, CHUNK * 4)]),
        ]

    def _out_copies(c, s):
        batch0 = (base0 + c * CHUNK) // N_MOD
        return [
            (obuf.at[pl.ds(s * KKM, KKM)],
             out_hbm.at[pl.ds(batch0 * M_PAD, KKM)])
        ]

    def start_in(c, s):
        for src, dst in _in_copies(c, s):
            pltpu.async_copy(src, dst, sem_in.at[s])

    def wait_in(c, s):
        for src, dst in _in_copies(c, s):
            pltpu.make_async_copy(src, dst, sem_in.at[s]).wait()

    def start_out(c, s):
        for src, dst in _out_copies(c, s):
            pltpu.async_copy(src, dst, sem_out.at[s])

    def wait_out(c, s):
        for src, dst in _out_copies(c, s):
            pltpu.make_async_copy(src, dst, sem_out.at[s]).wait()

    def compute(s):
        def m_body(m, carry_m):
            pe_v = [pebuf[m, pl.ds(16 * j, LANES)] for j in range(D // LANES)]
            for k in range(KK):
                r = m + N_MOD * k
                idxv = plsc.load_gather(
                    idxbuf, [jnp.full((LANES,), s * CHUNK + r, jnp.int32)])
                t = [plsc.load_gather(
                        tbuf, [jnp.full((LANES,), s * CHUNK * 4 + 4 * r + q,
                                        jnp.int32)])
                     for q in range(4)]
                for j in range(D // LANES):
                    g = plsc.load_gather(embbuf, [idxv, lane + 16 * j])
                    acc = g + pe_v[j]
                    acc = acc + t[0] * wv[0][j]
                    acc = acc + t[1] * wv[1][j]
                    acc = acc + t[2] * wv[2][j]
                    acc = acc + t[3] * wv[3][j]
                    acc = acc + xbuf[s * CHUNK + r, pl.ds(16 * j, LANES)]
                    obuf[s * KKM + k * M_PAD + m, pl.ds(16 * j, LANES)] = acc
            return carry_m

        lax.fori_loop(0, N_MOD, m_body, 0)

    start_in(0, 0)

    def chunk_body(c, carry):
        slot = lax.rem(c, 2)
        nxt = 1 - slot

        @pl.when(c + 1 < nchunks)
        def _prefetch():
            @pl.when(c >= 1)
            def _drain_prev_write():
                wait_out(c - 1, nxt)
            start_in(c + 1, nxt)

        wait_in(c, slot)
        compute(slot)
        start_out(c, slot)
        return carry

    lax.fori_loop(0, nchunks, chunk_body, 0)
    wait_out(nchunks - 2, lax.rem(nchunks - 2, 2))
    wait_out(nchunks - 1, lax.rem(nchunks - 1, 2))


def _sc_part(x3, idx2, types3, emb_table, pe_plus, W_type):
    batch, n_mod, d_model = x3.shape
    rows = batch * n_mod
    rows_per_worker = rows // N_WORKERS
    assert rows_per_worker * N_WORKERS == rows
    nchunks = rows_per_worker // CHUNK
    assert nchunks * CHUNK == rows_per_worker

    x2 = x3.reshape(rows, d_model)
    idx = idx2.reshape(rows)
    types_flat = types3.reshape(rows * 4)

    grid_kernel = pl.kernel(
        functools.partial(_sc_body, nchunks),
        out_type=jax.ShapeDtypeStruct((batch * M_PAD, d_model), jnp.float32),
        mesh=plsc.VectorSubcoreMesh(core_axis_name="c", subcore_axis_name="s"),
        scratch_types=[
            pltpu.VMEM((2 * CHUNK, d_model), jnp.float32),
            pltpu.VMEM((2 * KKM, d_model), jnp.float32),
            pltpu.VMEM((2 * CHUNK,), jnp.int32),
            pltpu.VMEM((2 * CHUNK * 4,), jnp.float32),
            pltpu.VMEM(emb_table.shape, jnp.float32),
            pltpu.VMEM((n_mod, d_model), jnp.float32),
            pltpu.VMEM((4, d_model), jnp.float32),
            pltpu.SemaphoreType.DMA((2,)),
            pltpu.SemaphoreType.DMA((2,)),
        ],
        compiler_params=pltpu.CompilerParams(needs_layout_passes=False),
    )
    return grid_kernel(x2, idx, types_flat, emb_table, pe_plus, W_type)


def _tc_body(x_ref, idx_ref, tp_ref, ohm_ref, bm_ref, sc_ref, o_ref):
    i = pl.program_id(0)

    @pl.when(i < NSC_BLOCKS)
    def _passthrough():
        sc = sc_ref[...].reshape(TB, M_PAD, D)
        o_ref[:, 0:24, :] = sc[:, 0:24, :]
        o_ref[:, 24, :] = sc[:, 24, :]
        o_ref[:, 25, :] = sc[:, 25, :]

    @pl.when(i >= NSC_BLOCKS)
    def _compute():
        idx2d = idx_ref[...].reshape(M_PAD, TB * M_PAD // M_PAD)
        pieces = []
        for g in range(M_PAD):
            bc = jnp.broadcast_to(idx2d[g][None, :], (56, TB))
            pieces.append((bc == lax.broadcasted_iota(
                jnp.int32, (56, TB), 0)).astype(jnp.float32))
        oh_t = jnp.concatenate(pieces, axis=1)

        cdims = (((0,), (0,)), ((), ()))
        ad = lax.dot_general(oh_t, bm_ref[0:56, :], cdims,
                             preferred_element_type=jnp.float32)
        ad = ad + lax.dot_general(ohm_ref[...], bm_ref[56:88, :], cdims,
                                  preferred_element_type=jnp.float32)
        ad = ad + lax.dot_general(tp_ref[...], bm_ref[88:96, :], cdims,
                                  preferred_element_type=jnp.float32)
        ad3 = ad.reshape(TB, M_PAD, D)
        x3 = x_ref[...]
        o_ref[:, 0:24, :] = x3[:, 0:24, :] + ad3[:, 0:24, :]
        o_ref[:, 24, :] = x3[:, 24, :] + ad3[:, 24, :]
        o_ref[:, 25, :] = x3[:, 25, :] + ad3[:, 25, :]


def _tc_assemble(x, idx_pad, tp, ohm, bigmat, sc_out):
    batch = x.shape[0]
    grid = (batch // TB,)
    nsc = NSC_BLOCKS
    rpb = TB * M_PAD
    return pl.pallas_call(
        _tc_body,
        grid=grid,
        in_specs=[
            pl.BlockSpec((TB, N_MOD, D), lambda i: (jnp.maximum(i, nsc), 0, 0)),
            pl.BlockSpec((rpb,), lambda i: (jnp.maximum(i, nsc),)),
            pl.BlockSpec((8, rpb), lambda i: (0, jnp.maximum(i, nsc))),
            pl.BlockSpec((M_PAD, rpb), lambda i: (0, 0)),
            pl.BlockSpec((96, D), lambda i: (0, 0)),
            pl.BlockSpec((rpb, D), lambda i: (jnp.minimum(i, nsc - 1), 0)),
        ],
        out_specs=pl.BlockSpec((TB, N_MOD, D), lambda i: (i, 0, 0)),
        out_shape=jax.ShapeDtypeStruct((batch, N_MOD, D), jnp.float32),
    )(x, idx_pad, tp, ohm, bigmat, sc_out)


def _make_pe_plus(n_mod, d_model, b_type):
    position = jnp.arange(0, n_mod, dtype=jnp.float32)[:, None]
    div_term = jnp.exp(jnp.arange(0, d_model, 2, dtype=jnp.float32)
                       * (-(jnp.log(10000.0) / d_model)))
    pe = jnp.zeros((n_mod, d_model), dtype=jnp.float32)
    pe = pe.at[:, 0::2].set(jnp.sin(position * div_term))
    pe = pe.at[:, 1::2].set(jnp.cos(position * div_term))
    return pe + b_type[None, :]


def kernel(x, modality_indices, modality_types, emb_table, W_type, b_type):
    batch, n_mod, d_model = x.shape
    assert n_mod == N_MOD and d_model == D
    pe_plus = _make_pe_plus(n_mod, d_model, b_type)
    idx_all = modality_indices.astype(jnp.int32)

    idx_pad = jnp.pad(idx_all, ((0, 0), (0, M_PAD - N_MOD))).reshape(
        batch * M_PAD)
    tp = jnp.pad(jnp.transpose(modality_types, (2, 0, 1)),
                 ((0, 4), (0, 0), (0, M_PAD - N_MOD))).reshape(
        8, batch * M_PAD)
    r_mod = jnp.remainder(jnp.arange(TB * M_PAD, dtype=jnp.int32), M_PAD)
    ohm = (r_mod[None, :] == jnp.arange(M_PAD, dtype=jnp.int32)[:, None]
           ).astype(jnp.float32)
    bigmat = jnp.zeros((96, D), jnp.float32)
    bigmat = bigmat.at[0:V].set(emb_table)
    bigmat = bigmat.at[56:56 + N_MOD].set(pe_plus)
    bigmat = bigmat.at[88:92].set(W_type)

    sc_out = _sc_part(x[:B_SC], idx_all[:B_SC], modality_types[:B_SC],
                      emb_table, pe_plus, W_type)
    return _tc_assemble(x, idx_pad, tp, ohm, bigmat, sc_out)


# speedup vs baseline: 3.2904x; 1.0435x over previous
import functools

import jax
import jax.numpy as jnp
from jax import lax
from jax.experimental import pallas as pl
from jax.experimental.pallas import tpu as pltpu
from jax.experimental.pallas import tpu_sc as plsc

D = 128
N_MOD = 26
M_PAD = 32
V = 50
LANES = 16
N_WORKERS = 32
KK = 4
CHUNK = N_MOD * KK
KKM = KK * M_PAD

B_SC = 2048
TB = 256
NSC_BLOCKS = B_SC // TB


def _sc_body(nchunks, x_hbm, idx_hbm, types_hbm, emb_hbm, pe_hbm,
             w_hbm, out_hbm, xbuf, obuf, idxbuf, tbuf, embbuf, pebuf, wbuf,
             sem_in, sem_out):
    wid = lax.axis_index("s") * 2 + lax.axis_index("c")
    base0 = wid * nchunks * CHUNK

    pltpu.sync_copy(emb_hbm, embbuf)
    pltpu.sync_copy(pe_hbm, pebuf)
    pltpu.sync_copy(w_hbm, wbuf)

    wv = [[wbuf[k, pl.ds(16 * j, LANES)] for j in range(D // LANES)]
          for k in range(4)]
    lane = lax.iota(jnp.int32, 16)

    def _in_copies(c, s):
        base = base0 + c * CHUNK
        return [
            (x_hbm.at[pl.ds(base, CHUNK)],
             xbuf.at[pl.ds(s * CHUNK, CHUNK)]),
            (idx_hbm.at[pl.ds(base, CHUNK)],
             idxbuf.at[pl.ds(s * CHUNK, CHUNK)]),
            (types_hbm.at[pl.ds(base * 4, CHUNK * 4)],
             tbuf.at[pl.ds(s * CHUNK * 4, CHUNK * 4)]),
        ]

    def _out_copies(c, s):
        batch0 = (base0 + c * CHUNK) // N_MOD
        return [
            (obuf.at[pl.ds(s * KKM, KKM)],
             out_hbm.at[pl.ds(batch0 * M_PAD, KKM)])
        ]

    def start_in(c, s):
        for src, dst in _in_copies(c, s):
            pltpu.async_copy(src, dst, sem_in.at[s])

    def wait_in(c, s):
        for src, dst in _in_copies(c, s):
            pltpu.make_async_copy(src, dst, sem_in.at[s]).wait()

    def start_out(c, s):
        for src, dst in _out_copies(c, s):
            pltpu.async_copy(src, dst, sem_out.at[s])

    def wait_out(c, s):
        for src, dst in _out_copies(c, s):
            pltpu.make_async_copy(src, dst, sem_out.at[s]).wait()

    def compute(s):
        def m_body(m, carry_m):
            pe_v = [pebuf[m, pl.ds(16 * j, LANES)] for j in range(D // LANES)]
            for k in range(KK):
                r = m + N_MOD * k
                idxv = plsc.load_gather(
                    idxbuf, [jnp.full((LANES,), s * CHUNK + r, jnp.int32)])
                t = [plsc.load_gather(
                        tbuf, [jnp.full((LANES,), s * CHUNK * 4 + 4 * r + q,
                                        jnp.int32)])
                     for q in range(4)]
                for j in range(D // LANES):
                    g = plsc.load_gather(embbuf, [idxv, lane + 16 * j])
                    acc = g + pe_v[j]
                    acc = acc + t[0] * wv[0][j]
                    acc = acc + t[1] * wv[1][j]
                    acc = acc + t[2] * wv[2][j]
                    acc = acc + t[3] * wv[3][j]
                    acc = acc + xbuf[s * CHUNK + r, pl.ds(16 * j, LANES)]
                    obuf[s * KKM + k * M_PAD + m, pl.ds(16 * j, LANES)] = acc
            return carry_m

        lax.fori_loop(0, N_MOD, m_body, 0)

    start_in(0, 0)

    def chunk_body(c, carry):
        slot = lax.rem(c, 2)
        nxt = 1 - slot

        @pl.when(c + 1 < nchunks)
        def _prefetch():
            @pl.when(c >= 1)
            def _drain_prev_write():
                wait_out(c - 1, nxt)
            start_in(c + 1, nxt)

        wait_in(c, slot)
        compute(slot)
        start_out(c, slot)
        return carry

    lax.fori_loop(0, nchunks, chunk_body, 0)
    wait_out(nchunks - 2, lax.rem(nchunks - 2, 2))
    wait_out(nchunks - 1, lax.rem(nchunks - 1, 2))


def _sc_part(x3, idx2, types3, emb_table, pe_plus, W_type):
    batch, n_mod, d_model = x3.shape
    rows = batch * n_mod
    rows_per_worker = rows // N_WORKERS
    assert rows_per_worker * N_WORKERS == rows
    nchunks = rows_per_worker // CHUNK
    assert nchunks * CHUNK == rows_per_worker

    x2 = x3.reshape(rows, d_model)
    idx = idx2.reshape(rows)
    types_flat = types3.reshape(rows * 4)

    grid_kernel = pl.kernel(
        functools.partial(_sc_body, nchunks),
        out_type=jax.ShapeDtypeStruct((batch * M_PAD, d_model), jnp.float32),
        mesh=plsc.VectorSubcoreMesh(core_axis_name="c", subcore_axis_name="s"),
        scratch_types=[
            pltpu.VMEM((2 * CHUNK, d_model), jnp.float32),
            pltpu.VMEM((2 * KKM, d_model), jnp.float32),
            pltpu.VMEM((2 * CHUNK,), jnp.int32),
            pltpu.VMEM((2 * CHUNK * 4,), jnp.float32),
            pltpu.VMEM(emb_table.shape, jnp.float32),
            pltpu.VMEM((n_mod, d_model), jnp.float32),
            pltpu.VMEM((4, d_model), jnp.float32),
            pltpu.SemaphoreType.DMA((2,)),
            pltpu.SemaphoreType.DMA((2,)),
        ],
        compiler_params=pltpu.CompilerParams(needs_layout_passes=False),
    )
    return grid_kernel(x2, idx, types_flat, emb_table, pe_plus, W_type)


def _tc_body(x_ref, idx_ref, tp_ref, ohm_ref, bm_ref, sc_ref, o_ref):
    i = pl.program_id(0)

    @pl.when(i < NSC_BLOCKS)
    def _passthrough():
        sc = sc_ref[...].reshape(TB, M_PAD, D)
        o_ref[:, 0:24, :] = sc[:, 0:24, :]
        o_ref[:, 24, :] = sc[:, 24, :]
        o_ref[:, 25, :] = sc[:, 25, :]

    @pl.when(i >= NSC_BLOCKS)
    def _compute():
        idx2d = idx_ref[...].reshape(M_PAD, TB * M_PAD // M_PAD)
        pieces = []
        for g in range(M_PAD):
            bc = jnp.broadcast_to(idx2d[g][None, :], (56, TB))
            pieces.append((bc == lax.broadcasted_iota(
                jnp.int32, (56, TB), 0)).astype(jnp.float32))
        oh_t = jnp.concatenate(pieces, axis=1)

        cdims = (((0,), (0,)), ((), ()))
        ad = lax.dot_general(oh_t, bm_ref[0:56, :], cdims,
                             preferred_element_type=jnp.float32)
        ad = ad + lax.dot_general(ohm_ref[...], bm_ref[56:88, :], cdims,
                                  preferred_element_type=jnp.float32)
        ad = ad + lax.dot_general(tp_ref[...], bm_ref[88:96, :], cdims,
                                  preferred_element_type=jnp.float32)
        ad3 = ad.reshape(TB, M_PAD, D)
        x3 = x_ref[...]
        o_ref[:, 0:24, :] = x3[:, 0:24, :] + ad3[:, 0:24, :]
        o_ref[:, 24, :] = x3[:, 24, :] + ad3[:, 24, :]
        o_ref[:, 25, :] = x3[:, 25, :] + ad3[:, 25, :]


def _tc_assemble(x, idx_pad, tp, ohm, bigmat, sc_out):
    batch = x.shape[0]
    grid = (batch // TB,)
    nsc = NSC_BLOCKS
    rpb = TB * M_PAD
    return pl.pallas_call(
        _tc_body,
        grid=grid,
        in_specs=[
            pl.BlockSpec((TB, N_MOD, D), lambda i: (jnp.maximum(i, nsc), 0, 0)),
            pl.BlockSpec((rpb,), lambda i: (jnp.maximum(i, nsc),)),
            pl.BlockSpec((8, rpb), lambda i: (0, jnp.maximum(i, nsc))),
            pl.BlockSpec((M_PAD, rpb), lambda i: (0, 0)),
            pl.BlockSpec((96, D), lambda i: (0, 0)),
            pl.BlockSpec((rpb, D), lambda i: (jnp.minimum(i, nsc - 1), 0)),
        ],
        out_specs=pl.BlockSpec((TB, N_MOD, D), lambda i: (i, 0, 0)),
        out_shape=jax.ShapeDtypeStruct((batch, N_MOD, D), jnp.float32),
    )(x, idx_pad, tp, ohm, bigmat, sc_out)


def _make_pe_plus(n_mod, d_model, b_type):
    position = jnp.arange(0, n_mod, dtype=jnp.float32)[:, None]
    div_term = jnp.exp(jnp.arange(0, d_model, 2, dtype=jnp.float32)
                       * (-(jnp.log(10000.0) / d_model)))
    pe = jnp.zeros((n_mod, d_model), dtype=jnp.float32)
    pe = pe.at[:, 0::2].set(jnp.sin(position * div_term))
    pe = pe.at[:, 1::2].set(jnp.cos(position * div_term))
    return pe + b_type[None, :]


def kernel(x, modality_indices, modality_types, emb_table, W_type, b_type):
    batch, n_mod, d_model = x.shape
    assert n_mod == N_MOD and d_model == D
    pe_plus = _make_pe_plus(n_mod, d_model, b_type)
    idx_all = modality_indices.astype(jnp.int32)

    idx_pad = jnp.pad(idx_all, ((0, 0), (0, M_PAD - N_MOD))).reshape(
        batch * M_PAD)
    tp = jnp.pad(jnp.transpose(modality_types, (2, 0, 1)),
                 ((0, 4), (0, 0), (0, M_PAD - N_MOD))).reshape(
        8, batch * M_PAD)
    r_mod = jnp.remainder(jnp.arange(TB * M_PAD, dtype=jnp.int32), M_PAD)
    ohm = (r_mod[None, :] == jnp.arange(M_PAD, dtype=jnp.int32)[:, None]
           ).astype(jnp.float32)
    bigmat = jnp.zeros((96, D), jnp.float32)
    bigmat = bigmat.at[0:V].set(emb_table)
    bigmat = bigmat.at[56:56 + N_MOD].set(pe_plus)
    bigmat = bigmat.at[88:92].set(W_type)

    sc_out = _sc_part(x[:B_SC], idx_all[:B_SC], modality_types[:B_SC],
                      emb_table, pe_plus, W_type)
    return _tc_assemble(x, idx_pad, tp, ohm, bigmat, sc_out)


# speedup vs baseline: 3.4976x; 1.0630x over previous
import functools

import jax
import jax.numpy as jnp
from jax import lax
from jax.experimental import pallas as pl
from jax.experimental.pallas import tpu as pltpu
from jax.experimental.pallas import tpu_sc as plsc

D = 128
N_MOD = 26
M_PAD = 32
V = 50
LANES = 16
N_WORKERS = 32
KK = 4
CHUNK = N_MOD * KK
KKM = KK * M_PAD

B_SC = 1024
TB = 256
NSC_BLOCKS = B_SC // TB


def _sc_body(nchunks, x_hbm, idx_hbm, types_hbm, emb_hbm, pe_hbm,
             w_hbm, out_hbm, xbuf, obuf, idxbuf, tbuf, embbuf, pebuf, wbuf,
             sem_in, sem_out):
    wid = lax.axis_index("s") * 2 + lax.axis_index("c")
    base0 = wid * nchunks * CHUNK

    pltpu.sync_copy(emb_hbm, embbuf)
    pltpu.sync_copy(pe_hbm, pebuf)
    pltpu.sync_copy(w_hbm, wbuf)

    wv = [[wbuf[k, pl.ds(16 * j, LANES)] for j in range(D // LANES)]
          for k in range(4)]
    lane = lax.iota(jnp.int32, 16)

    def _in_copies(c, s):
        base = base0 + c * CHUNK
        return [
            (x_hbm.at[pl.ds(base, CHUNK)],
             xbuf.at[pl.ds(s * CHUNK, CHUNK)]),
            (idx_hbm.at[pl.ds(base, CHUNK)],
             idxbuf.at[pl.ds(s * CHUNK, CHUNK)]),
            (types_hbm.at[pl.ds(base * 4, CHUNK * 4)],
             tbuf.at[pl.ds(s * CHUNK * 4, CHUNK * 4)]),
        ]

    def _out_copies(c, s):
        batch0 = (base0 + c * CHUNK) // N_MOD
        return [
            (obuf.at[pl.ds(s * KKM, KKM)],
             out_hbm.at[pl.ds(batch0 * M_PAD, KKM)])
        ]

    def start_in(c, s):
        for src, dst in _in_copies(c, s):
            pltpu.async_copy(src, dst, sem_in.at[s])

    def wait_in(c, s):
        for src, dst in _in_copies(c, s):
            pltpu.make_async_copy(src, dst, sem_in.at[s]).wait()

    def start_out(c, s):
        for src, dst in _out_copies(c, s):
            pltpu.async_copy(src, dst, sem_out.at[s])

    def wait_out(c, s):
        for src, dst in _out_copies(c, s):
            pltpu.make_async_copy(src, dst, sem_out.at[s]).wait()

    def compute(s):
        def m_body(m, carry_m):
            pe_v = [pebuf[m, pl.ds(16 * j, LANES)] for j in range(D // LANES)]
            for k in range(KK):
                r = m + N_MOD * k
                idxv = plsc.load_gather(
                    idxbuf, [jnp.full((LANES,), s * CHUNK + r, jnp.int32)])
                t = [plsc.load_gather(
                        tbuf, [jnp.full((LANES,), s * CHUNK * 4 + 4 * r + q,
                                        jnp.int32)])
                     for q in range(4)]
                for j in range(D // LANES):
                    g = plsc.load_gather(embbuf, [idxv, lane + 16 * j])
                    acc = g + pe_v[j]
                    acc = acc + t[0] * wv[0][j]
                    acc = acc + t[1] * wv[1][j]
                    acc = acc + t[2] * wv[2][j]
                    acc = acc + t[3] * wv[3][j]
                    acc = acc + xbuf[s * CHUNK + r, pl.ds(16 * j, LANES)]
                    obuf[s * KKM + k * M_PAD + m, pl.ds(16 * j, LANES)] = acc
            return carry_m

        lax.fori_loop(0, N_MOD, m_body, 0)

    start_in(0, 0)

    def chunk_body(c, carry):
        slot = lax.rem(c, 2)
        nxt = 1 - slot

        @pl.when(c + 1 < nchunks)
        def _prefetch():
            @pl.when(c >= 1)
            def _drain_prev_write():
                wait_out(c - 1, nxt)
            start_in(c + 1, nxt)

        wait_in(c, slot)
        compute(slot)
        start_out(c, slot)
        return carry

    lax.fori_loop(0, nchunks, chunk_body, 0)
    wait_out(nchunks - 2, lax.rem(nchunks - 2, 2))
    wait_out(nchunks - 1, lax.rem(nchunks - 1, 2))


def _sc_part(x3, idx2, types3, emb_table, pe_plus, W_type):
    batch, n_mod, d_model = x3.shape
    rows = batch * n_mod
    rows_per_worker = rows // N_WORKERS
    assert rows_per_worker * N_WORKERS == rows
    nchunks = rows_per_worker // CHUNK
    assert nchunks * CHUNK == rows_per_worker

    x2 = x3.reshape(rows, d_model)
    idx = idx2.reshape(rows)
    types_flat = types3.reshape(rows * 4)

    grid_kernel = pl.kernel(
        functools.partial(_sc_body, nchunks),
        out_type=jax.ShapeDtypeStruct((batch * M_PAD, d_model), jnp.float32),
        mesh=plsc.VectorSubcoreMesh(core_axis_name="c", subcore_axis_name="s"),
        scratch_types=[
            pltpu.VMEM((2 * CHUNK, d_model), jnp.float32),
            pltpu.VMEM((2 * KKM, d_model), jnp.float32),
            pltpu.VMEM((2 * CHUNK,), jnp.int32),
            pltpu.VMEM((2 * CHUNK * 4,), jnp.float32),
            pltpu.VMEM(emb_table.shape, jnp.float32),
            pltpu.VMEM((n_mod, d_model), jnp.float32),
            pltpu.VMEM((4, d_model), jnp.float32),
            pltpu.SemaphoreType.DMA((2,)),
            pltpu.SemaphoreType.DMA((2,)),
        ],
        compiler_params=pltpu.CompilerParams(needs_layout_passes=False),
    )
    return grid_kernel(x2, idx, types_flat, emb_table, pe_plus, W_type)


def _tc_body(x_ref, idx_ref, tp_ref, ohm_ref, bm_ref, sc_ref, o_ref):
    i = pl.program_id(0)

    @pl.when(i < NSC_BLOCKS)
    def _passthrough():
        sc = sc_ref[...].reshape(TB, M_PAD, D)
        o_ref[:, 0:24, :] = sc[:, 0:24, :]
        o_ref[:, 24, :] = sc[:, 24, :]
        o_ref[:, 25, :] = sc[:, 25, :]

    @pl.when(i >= NSC_BLOCKS)
    def _compute():
        idx2d = idx_ref[...].reshape(M_PAD, TB * M_PAD // M_PAD)
        pieces = []
        for g in range(M_PAD):
            bc = jnp.broadcast_to(idx2d[g][None, :], (56, TB))
            pieces.append((bc == lax.broadcasted_iota(
                jnp.int32, (56, TB), 0)).astype(jnp.float32))
        oh_t = jnp.concatenate(pieces, axis=1)

        cdims = (((0,), (0,)), ((), ()))
        ad = lax.dot_general(oh_t, bm_ref[0:56, :], cdims,
                             preferred_element_type=jnp.float32)
        ad = ad + lax.dot_general(ohm_ref[...], bm_ref[56:88, :], cdims,
                                  preferred_element_type=jnp.float32)
        ad = ad + lax.dot_general(tp_ref[...], bm_ref[88:96, :], cdims,
                                  preferred_element_type=jnp.float32)
        ad3 = ad.reshape(TB, M_PAD, D)
        x3 = x_ref[...]
        o_ref[:, 0:24, :] = x3[:, 0:24, :] + ad3[:, 0:24, :]
        o_ref[:, 24, :] = x3[:, 24, :] + ad3[:, 24, :]
        o_ref[:, 25, :] = x3[:, 25, :] + ad3[:, 25, :]


def _tc_assemble(x, idx_pad, tp, ohm, bigmat, sc_out):
    batch = x.shape[0]
    grid = (batch // TB,)
    nsc = NSC_BLOCKS
    rpb = TB * M_PAD
    return pl.pallas_call(
        _tc_body,
        grid=grid,
        in_specs=[
            pl.BlockSpec((TB, N_MOD, D), lambda i: (jnp.maximum(i, nsc), 0, 0)),
            pl.BlockSpec((rpb,), lambda i: (jnp.maximum(i, nsc),)),
            pl.BlockSpec((8, rpb), lambda i: (0, jnp.maximum(i, nsc))),
            pl.BlockSpec((M_PAD, rpb), lambda i: (0, 0)),
            pl.BlockSpec((96, D), lambda i: (0, 0)),
            pl.BlockSpec((rpb, D), lambda i: (jnp.minimum(i, nsc - 1), 0)),
        ],
        out_specs=pl.BlockSpec((TB, N_MOD, D), lambda i: (i, 0, 0)),
        out_shape=jax.ShapeDtypeStruct((batch, N_MOD, D), jnp.float32),
    )(x, idx_pad, tp, ohm, bigmat, sc_out)


def _make_pe_plus(n_mod, d_model, b_type):
    position = jnp.arange(0, n_mod, dtype=jnp.float32)[:, None]
    div_term = jnp.exp(jnp.arange(0, d_model, 2, dtype=jnp.float32)
                       * (-(jnp.log(10000.0) / d_model)))
    pe = jnp.zeros((n_mod, d_model), dtype=jnp.float32)
    pe = pe.at[:, 0::2].set(jnp.sin(position * div_term))
    pe = pe.at[:, 1::2].set(jnp.cos(position * div_term))
    return pe + b_type[None, :]


def kernel(x, modality_indices, modality_types, emb_table, W_type, b_type):
    batch, n_mod, d_model = x.shape
    assert n_mod == N_MOD and d_model == D
    pe_plus = _make_pe_plus(n_mod, d_model, b_type)
    idx_all = modality_indices.astype(jnp.int32)

    idx_pad = jnp.pad(idx_all, ((0, 0), (0, M_PAD - N_MOD))).reshape(
        batch * M_PAD)
    tp = jnp.pad(jnp.transpose(modality_types, (2, 0, 1)),
                 ((0, 4), (0, 0), (0, M_PAD - N_MOD))).reshape(
        8, batch * M_PAD)
    r_mod = jnp.remainder(jnp.arange(TB * M_PAD, dtype=jnp.int32), M_PAD)
    ohm = (r_mod[None, :] == jnp.arange(M_PAD, dtype=jnp.int32)[:, None]
           ).astype(jnp.float32)
    bigmat = jnp.zeros((96, D), jnp.float32)
    bigmat = bigmat.at[0:V].set(emb_table)
    bigmat = bigmat.at[56:56 + N_MOD].set(pe_plus)
    bigmat = bigmat.at[88:92].set(W_type)

    sc_out = _sc_part(x[:B_SC], idx_all[:B_SC], modality_types[:B_SC],
                      emb_table, pe_plus, W_type)
    return _tc_assemble(x, idx_pad, tp, ohm, bigmat, sc_out)


# speedup vs baseline: 3.6811x; 1.0525x over previous
import functools

import jax
import jax.numpy as jnp
from jax import lax
from jax.experimental import pallas as pl
from jax.experimental.pallas import tpu as pltpu
from jax.experimental.pallas import tpu_sc as plsc

D = 128
N_MOD = 26
M_PAD = 32
V = 50
LANES = 16
N_WORKERS = 32
KK = 4
CHUNK = N_MOD * KK
KKM = KK * M_PAD

B_SC = 1024
TB = 256
NSC_BLOCKS = B_SC // TB


def _sc_body(nchunks, x_hbm, idx_hbm, types_hbm, emb_hbm, pe_hbm,
             w_hbm, out_hbm, xbuf, obuf, idxbuf, tbuf, embbuf, pebuf, wbuf,
             sem_in, sem_out):
    wid = lax.axis_index("s") * 2 + lax.axis_index("c")
    base0 = wid * nchunks * CHUNK

    pltpu.sync_copy(emb_hbm, embbuf)
    pltpu.sync_copy(pe_hbm, pebuf)
    pltpu.sync_copy(w_hbm, wbuf)

    wv = [[wbuf[k, pl.ds(16 * j, LANES)] for j in range(D // LANES)]
          for k in range(4)]
    lane = lax.iota(jnp.int32, 16)

    def _in_copies(c, s):
        base = base0 + c * CHUNK
        return [
            (x_hbm.at[pl.ds(base, CHUNK)],
             xbuf.at[pl.ds(s * CHUNK, CHUNK)]),
            (idx_hbm.at[pl.ds(base, CHUNK)],
             idxbuf.at[pl.ds(s * CHUNK, CHUNK)]),
            (types_hbm.at[pl.ds(base * 4, CHUNK * 4)],
             tbuf.at[pl.ds(s * CHUNK * 4, CHUNK * 4)]),
        ]

    def _out_copies(c, s):
        batch0 = (base0 + c * CHUNK) // N_MOD
        return [
            (obuf.at[pl.ds(s * KKM, KKM)],
             out_hbm.at[pl.ds(batch0 * M_PAD, KKM)])
        ]

    def start_in(c, s):
        for src, dst in _in_copies(c, s):
            pltpu.async_copy(src, dst, sem_in.at[s])

    def wait_in(c, s):
        for src, dst in _in_copies(c, s):
            pltpu.make_async_copy(src, dst, sem_in.at[s]).wait()

    def start_out(c, s):
        for src, dst in _out_copies(c, s):
            pltpu.async_copy(src, dst, sem_out.at[s])

    def wait_out(c, s):
        for src, dst in _out_copies(c, s):
            pltpu.make_async_copy(src, dst, sem_out.at[s]).wait()

    def compute(s):
        def m_body(m, carry_m):
            pe_v = [pebuf[m, pl.ds(16 * j, LANES)] for j in range(D // LANES)]
            for k in range(KK):
                r = m + N_MOD * k
                idxv = plsc.load_gather(
                    idxbuf, [jnp.full((LANES,), s * CHUNK + r, jnp.int32)])
                t = [plsc.load_gather(
                        tbuf, [jnp.full((LANES,), s * CHUNK * 4 + 4 * r + q,
                                        jnp.int32)])
                     for q in range(4)]
                for j in range(D // LANES):
                    g = plsc.load_gather(embbuf, [idxv, lane + 16 * j])
                    acc = g + pe_v[j]
                    acc = acc + t[0] * wv[0][j]
                    acc = acc + t[1] * wv[1][j]
                    acc = acc + t[2] * wv[2][j]
                    acc = acc + t[3] * wv[3][j]
                    acc = acc + xbuf[s * CHUNK + r, pl.ds(16 * j, LANES)]
                    obuf[s * KKM + k * M_PAD + m, pl.ds(16 * j, LANES)] = acc
            return carry_m

        lax.fori_loop(0, N_MOD, m_body, 0)

    start_in(0, 0)

    def chunk_body(c, carry):
        slot = lax.rem(c, 2)
        nxt = 1 - slot

        @pl.when(c + 1 < nchunks)
        def _prefetch():
            @pl.when(c >= 1)
            def _drain_prev_write():
                wait_out(c - 1, nxt)
            start_in(c + 1, nxt)

        wait_in(c, slot)
        compute(slot)
        start_out(c, slot)
        return carry

    lax.fori_loop(0, nchunks, chunk_body, 0)
    wait_out(nchunks - 2, lax.rem(nchunks - 2, 2))
    wait_out(nchunks - 1, lax.rem(nchunks - 1, 2))


def _sc_part(x3, idx2, types3, emb_table, pe_plus, W_type):
    batch, n_mod, d_model = x3.shape
    rows = batch * n_mod
    rows_per_worker = rows // N_WORKERS
    assert rows_per_worker * N_WORKERS == rows
    nchunks = rows_per_worker // CHUNK
    assert nchunks * CHUNK == rows_per_worker

    x2 = x3.reshape(rows, d_model)
    idx = idx2.reshape(rows)
    types_flat = types3.reshape(rows * 4)

    grid_kernel = pl.kernel(
        functools.partial(_sc_body, nchunks),
        out_type=jax.ShapeDtypeStruct((batch * M_PAD, d_model), jnp.float32),
        mesh=plsc.VectorSubcoreMesh(core_axis_name="c", subcore_axis_name="s"),
        scratch_types=[
            pltpu.VMEM((2 * CHUNK, d_model), jnp.float32),
            pltpu.VMEM((2 * KKM, d_model), jnp.float32),
            pltpu.VMEM((2 * CHUNK,), jnp.int32),
            pltpu.VMEM((2 * CHUNK * 4,), jnp.float32),
            pltpu.VMEM(emb_table.shape, jnp.float32),
            pltpu.VMEM((n_mod, d_model), jnp.float32),
            pltpu.VMEM((4, d_model), jnp.float32),
            pltpu.SemaphoreType.DMA((2,)),
            pltpu.SemaphoreType.DMA((2,)),
        ],
        compiler_params=pltpu.CompilerParams(needs_layout_passes=False),
    )
    return grid_kernel(x2, idx, types_flat, emb_table, pe_plus, W_type)


def _tc_body(x_ref, idx_ref, tp_ref, ohm_ref, bm_ref, sc_ref, o_ref):
    i = pl.program_id(0)

    @pl.when(i < NSC_BLOCKS)
    def _passthrough():
        sc = sc_ref[...].reshape(TB, M_PAD, D)
        o_ref[:, 0:24, :] = sc[:, 0:24, :]
        o_ref[:, 24, :] = sc[:, 24, :]
        o_ref[:, 25, :] = sc[:, 25, :]

    @pl.when(i >= NSC_BLOCKS)
    def _compute():
        idx2d = idx_ref[...].reshape(M_PAD, TB * M_PAD // M_PAD)
        pieces = []
        for g in range(M_PAD):
            bc = jnp.broadcast_to(idx2d[g][None, :], (56, TB))
            pieces.append((bc == lax.broadcasted_iota(
                jnp.int32, (56, TB), 0)).astype(jnp.float32))
        oh_t = jnp.concatenate(pieces, axis=1)

        a_t = jnp.concatenate([oh_t, ohm_ref[...], tp_ref[...]], axis=0)
        ad = lax.dot_general(a_t, bm_ref[...], (((0,), (0,)), ((), ())),
                             preferred_element_type=jnp.float32)
        ad3 = ad.reshape(TB, M_PAD, D)
        x3 = x_ref[...]
        o_ref[:, 0:24, :] = x3[:, 0:24, :] + ad3[:, 0:24, :]
        o_ref[:, 24, :] = x3[:, 24, :] + ad3[:, 24, :]
        o_ref[:, 25, :] = x3[:, 25, :] + ad3[:, 25, :]


def _tc_assemble(x, idx_pad, tp, ohm, bigmat, sc_out):
    batch = x.shape[0]
    grid = (batch // TB,)
    nsc = NSC_BLOCKS
    rpb = TB * M_PAD
    return pl.pallas_call(
        _tc_body,
        grid=grid,
        in_specs=[
            pl.BlockSpec((TB, N_MOD, D), lambda i: (jnp.maximum(i, nsc), 0, 0)),
            pl.BlockSpec((rpb,), lambda i: (jnp.maximum(i, nsc),)),
            pl.BlockSpec((8, rpb), lambda i: (0, jnp.maximum(i, nsc))),
            pl.BlockSpec((M_PAD, rpb), lambda i: (0, 0)),
            pl.BlockSpec((96, D), lambda i: (0, 0)),
            pl.BlockSpec((rpb, D), lambda i: (jnp.minimum(i, nsc - 1), 0)),
        ],
        out_specs=pl.BlockSpec((TB, N_MOD, D), lambda i: (i, 0, 0)),
        out_shape=jax.ShapeDtypeStruct((batch, N_MOD, D), jnp.float32),
    )(x, idx_pad, tp, ohm, bigmat, sc_out)


def _make_pe_plus(n_mod, d_model, b_type):
    position = jnp.arange(0, n_mod, dtype=jnp.float32)[:, None]
    div_term = jnp.exp(jnp.arange(0, d_model, 2, dtype=jnp.float32)
                       * (-(jnp.log(10000.0) / d_model)))
    pe = jnp.zeros((n_mod, d_model), dtype=jnp.float32)
    pe = pe.at[:, 0::2].set(jnp.sin(position * div_term))
    pe = pe.at[:, 1::2].set(jnp.cos(position * div_term))
    return pe + b_type[None, :]


def kernel(x, modality_indices, modality_types, emb_table, W_type, b_type):
    batch, n_mod, d_model = x.shape
    assert n_mod == N_MOD and d_model == D
    pe_plus = _make_pe_plus(n_mod, d_model, b_type)
    idx_all = modality_indices.astype(jnp.int32)

    idx_pad = jnp.pad(idx_all, ((0, 0), (0, M_PAD - N_MOD))).reshape(
        batch * M_PAD)
    tp = jnp.pad(jnp.transpose(modality_types, (2, 0, 1)),
                 ((0, 4), (0, 0), (0, M_PAD - N_MOD))).reshape(
        8, batch * M_PAD)
    r_mod = jnp.remainder(jnp.arange(TB * M_PAD, dtype=jnp.int32), M_PAD)
    ohm = (r_mod[None, :] == jnp.arange(M_PAD, dtype=jnp.int32)[:, None]
           ).astype(jnp.float32)
    bigmat = jnp.zeros((96, D), jnp.float32)
    bigmat = bigmat.at[0:V].set(emb_table)
    bigmat = bigmat.at[56:56 + N_MOD].set(pe_plus)
    bigmat = bigmat.at[88:92].set(W_type)

    sc_out = _sc_part(x[:B_SC], idx_all[:B_SC], modality_types[:B_SC],
                      emb_table, pe_plus, W_type)
    return _tc_assemble(x, idx_pad, tp, ohm, bigmat, sc_out)
